# baseline (speedup 1.0000x reference)
import jax
import jax.numpy as jnp
from jax.experimental import pallas as pl
from jax.experimental.pallas import tpu as pltpu

N = 100000
F = 128
BLOCK = 10000
NSPLIT = 2
BLOCKS_PER_SPLIT = N // BLOCK // NSPLIT


def _mlp_kernel(t_ref, ws0_ref, bs0_ref, wt0_ref, bt0_ref,
                ws1_ref, bs1_ref, wt1_ref, bt1_ref, out_ref,
                wmid_ref, bmid_ref):
    dims_nt = (((1,), (1,)), ((), ()))
    dims_nn = (((1,), (0,)), ((), ()))

    @pl.when(pl.program_id(0) == 0)
    def _prep():
        wmid_ref[...] = jax.lax.dot_general(
            ws1_ref[...], wt0_ref[...], dims_nn,
            preferred_element_type=jnp.float32)
        bmid_ref[...] = jax.lax.dot_general(
            bt0_ref[...], ws1_ref[...], dims_nt,
            preferred_element_type=jnp.float32) + bs1_ref[...]

    bf16 = jnp.bfloat16
    x = t_ref[...].astype(bf16)
    h = jax.lax.dot_general(x, ws0_ref[...].astype(bf16), dims_nt,
                            preferred_element_type=jnp.float32)
    h = jnp.maximum(h + bs0_ref[...], 0.0).astype(bf16)
    h = jax.lax.dot_general(h, wmid_ref[...].astype(bf16), dims_nt,
                            preferred_element_type=jnp.float32)
    h = jnp.maximum(h + bmid_ref[...], 0.0).astype(bf16)
    out_ref[...] = jax.lax.dot_general(h, wt1_ref[...].astype(bf16), dims_nt,
                                       preferred_element_type=jnp.float32) + bt1_ref[...]


def _mlp_kernel_acc(t_ref, ws0_ref, bs0_ref, wt0_ref, bt0_ref,
                    ws1_ref, bs1_ref, wt1_ref, bt1_ref, acc_ref, out_ref,
                    wmid_ref, bmid_ref):
    del acc_ref
    _mlp_kernel(t_ref, ws0_ref, bs0_ref, wt0_ref, bt0_ref,
                ws1_ref, bs1_ref, wt1_ref, bt1_ref, out_ref,
                wmid_ref, bmid_ref)


def _range_call(args, acc, base):
    weight_spec = pl.BlockSpec((F, F), lambda i: (0, 0))
    bias_spec = pl.BlockSpec((1, F), lambda i: (0, 0))
    row_spec = pl.BlockSpec((BLOCK, F), lambda i: (base + i, 0))
    in_specs = [
        row_spec,
        weight_spec, bias_spec,
        weight_spec, bias_spec,
        weight_spec, bias_spec,
        weight_spec, bias_spec,
    ]
    operands = list(args)
    if acc is None:
        body = _mlp_kernel
        aliases = {}
    else:
        body = _mlp_kernel_acc
        in_specs = in_specs + [pl.BlockSpec(memory_space=pl.ANY)]
        operands = operands + [acc]
        aliases = {9: 0}
    return pl.pallas_call(
        body,
        grid=(BLOCKS_PER_SPLIT,),
        in_specs=in_specs,
        out_specs=row_spec,
        out_shape=jax.ShapeDtypeStruct((N, F), jnp.float32),
        input_output_aliases=aliases,
        scratch_shapes=[
            pltpu.VMEM((F, F), jnp.float32),
            pltpu.VMEM((1, F), jnp.float32),
        ],
    )(*operands)


@jax.jit
def kernel(t, Ws0, bs0, Wt0, bt0, Ws1, bs1, Wt1, bt1):
    args = (t, Ws0, bs0.reshape(1, F), Wt0, bt0.reshape(1, F),
            Ws1, bs1.reshape(1, F), Wt1, bt1.reshape(1, F))
    out = _range_call(args, None, 0)
    for s in range(1, NSPLIT):
        out = _range_call(args, out, s * BLOCKS_PER_SPLIT)
    return out


# speedup vs baseline: 1.4289x; 1.4289x over previous
import jax
import jax.numpy as jnp
from jax.experimental import pallas as pl
from jax.experimental.pallas import tpu as pltpu

N = 100000
F = 128
BLOCK = 20000


def _fused_mlp_kernel(t_ref, ws0_ref, bs0_ref, wt0_ref, bt0_ref,
                      ws1_ref, bs1_ref, wt1_ref, bt1_ref, out_ref,
                      wmid_ref, bmid_ref):
    dims_nt = (((1,), (1,)), ((), ()))
    dims_nn = (((1,), (0,)), ((), ()))
    bf16 = jnp.bfloat16

    @pl.when(pl.program_id(0) == 0)
    def _prep():
        wmid_ref[...] = jax.lax.dot_general(
            ws1_ref[...], wt0_ref[...], dims_nn,
            preferred_element_type=jnp.float32)
        bmid_ref[...] = jax.lax.dot_general(
            bt0_ref[...], ws1_ref[...], dims_nt,
            preferred_element_type=jnp.float32) + bs1_ref[...]

    x = t_ref[...].astype(bf16)
    h = jax.lax.dot_general(x, ws0_ref[...].astype(bf16), dims_nt,
                            preferred_element_type=jnp.float32)
    h = jnp.maximum(h.astype(bf16) + bs0_ref[...].astype(bf16), 0.0)
    h = jax.lax.dot_general(h, wmid_ref[...].astype(bf16), dims_nt,
                            preferred_element_type=jnp.float32)
    h = jnp.maximum(h.astype(bf16) + bmid_ref[...].astype(bf16), 0.0)
    out_ref[...] = jax.lax.dot_general(h, wt1_ref[...].astype(bf16), dims_nt,
                                       preferred_element_type=jnp.float32) + bt1_ref[...]


@jax.jit
def kernel(t, Ws0, bs0, Wt0, bt0, Ws1, bs1, Wt1, bt1):
    weight_spec = pl.BlockSpec((F, F), lambda i: (0, 0))
    bias_spec = pl.BlockSpec((1, F), lambda i: (0, 0))
    grid = (N // BLOCK,)
    return pl.pallas_call(
        _fused_mlp_kernel,
        grid=grid,
        in_specs=[
            pl.BlockSpec((BLOCK, F), lambda i: (i, 0)),
            weight_spec, bias_spec,
            weight_spec, bias_spec,
            weight_spec, bias_spec,
            weight_spec, bias_spec,
        ],
        out_specs=pl.BlockSpec((BLOCK, F), lambda i: (i, 0)),
        out_shape=jax.ShapeDtypeStruct((N, F), jnp.float32),
        scratch_shapes=[
            pltpu.VMEM((F, F), jnp.float32),
            pltpu.VMEM((1, F), jnp.float32),
        ],
    )(t, Ws0, bs0.reshape(1, F), Wt0, bt0.reshape(1, F),
      Ws1, bs1.reshape(1, F), Wt1, bt1.reshape(1, F))
